# baseline (speedup 1.0000x reference)
import functools

import jax
import jax.numpy as jnp
from jax import lax
from jax.experimental import pallas as pl
from jax.experimental.pallas import tpu as pltpu
from jax.experimental.pallas import tpu_sc as plsc

N_BINS = 10000
ROW_W = 128
N_ROWS = 80
PAD_BINS = N_ROWS * ROW_W
N_SUBCORES = 16
N_REL = 4
E_PER_REL = 80000
E_TOTAL = 2 * N_REL * E_PER_REL
CHUNK = E_TOTAL // N_SUBCORES
VECS = CHUNK // 16
OUT_ROWS = 8
OUT_TILES = N_ROWS // OUT_ROWS


def _make_kernel():
    mesh = plsc.VectorSubcoreMesh(core_axis_name="c", subcore_axis_name="s",
                                  num_cores=1)

    @functools.partial(
        pl.kernel,
        out_type=jax.ShapeDtypeStruct((N_ROWS, ROW_W), jnp.float32),
        mesh=mesh,
        compiler_params=pltpu.CompilerParams(needs_layout_passes=False,
                                             use_tc_tiling_on_sc=False),
        scratch_types=[
            pltpu.VMEM((CHUNK,), jnp.int32),
            pltpu.VMEM((N_ROWS, ROW_W), jnp.float32),
            pltpu.VMEM((N_ROWS,), jnp.int32),
            pltpu.VMEM((OUT_ROWS, ROW_W), jnp.float32),
            pltpu.VMEM_SHARED((N_ROWS, ROW_W), jnp.float32),
            pltpu.SemaphoreType.DMA,
        ],
    )
    def hist_recip(idxa_hbm, idxb_hbm, out_hbm, idx_v, hist2_v, rowidx_v,
                   out_v, shared, sem):
        cid = lax.axis_index("c")
        sid = lax.axis_index("s")

        per_rel = E_PER_REL // N_SUBCORES
        cps = []
        for k, src_hbm in ((0, idxa_hbm), (1, idxb_hbm)):
            for r in range(N_REL):
                cps.append(pltpu.async_copy(
                    src_hbm.at[r, pl.ds(sid * per_rel, per_rel)],
                    idx_v.at[pl.ds((k * N_REL + r) * per_rel, per_rel)],
                    sem))

        zeros = jnp.zeros((16,), jnp.float32)

        def zero_body(i, _):
            hist2_v[i >> 3, pl.ds((i & 7) * 16, 16)] = zeros
            return 0

        lax.fori_loop(0, N_ROWS * 8, zero_body, 0, unroll=8)

        @pl.when(sid == 0)
        def _():
            pltpu.sync_copy(hist2_v, shared)

        base_iota = lax.iota(jnp.int32, 16)

        def rowidx_body(i, _):
            rowidx_v[pl.ds(i * 16, 16)] = base_iota + i * 16
            return 0

        lax.fori_loop(0, N_ROWS // 16, rowidx_body, 0)

        for cp in cps:
            cp.wait()

        ones = jnp.full((16,), 1.0, jnp.float32)

        def acc_body(i, _):
            v = idx_v[pl.ds(i * 16, 16)]
            plsc.addupdate_scatter(hist2_v, [v >> 7, v & 127], ones)
            return 0

        lax.fori_loop(0, VECS, acc_body, 0, unroll=10)

        plsc.subcore_barrier()

        pltpu.sync_copy(hist2_v, shared.at[rowidx_v], add=True)

        plsc.subcore_barrier()

        @pl.when((cid == 0) & (sid < OUT_TILES))
        def _():
            pltpu.sync_copy(shared.at[pl.ds(sid * OUT_ROWS, OUT_ROWS)],
                            out_v)
            for r in range(OUT_ROWS):
                for j in range(ROW_W // 16):
                    x = out_v[r, pl.ds(j * 16, 16)]
                    out_v[r, pl.ds(j * 16, 16)] = 1.0 / (x + 4.0)
            pltpu.sync_copy(out_v,
                            out_hbm.at[pl.ds(sid * OUT_ROWS, OUT_ROWS)])

    return hist_recip


_HIST_RECIP = _make_kernel()


def kernel(X, ref_a, ref_b):
    del X
    hist = _HIST_RECIP(ref_a, ref_b)
    return hist.reshape(PAD_BINS)[:N_BINS]


# speedup vs baseline: 1.1532x; 1.1532x over previous
import functools

import jax
import jax.numpy as jnp
from jax import lax
from jax.experimental import pallas as pl
from jax.experimental.pallas import tpu as pltpu
from jax.experimental.pallas import tpu_sc as plsc

N_BINS = 10000
ROW_W = 128
N_ROWS = 80
PAD_BINS = N_ROWS * ROW_W
N_TILES = 32
E_TOTAL = 8 * 80000
CHUNK = E_TOTAL // N_TILES
HALF = CHUNK // 2
VECS = CHUNK // 16


def _make_sc_hist():
    mesh = plsc.VectorSubcoreMesh(core_axis_name="c", subcore_axis_name="s")

    @functools.partial(
        pl.kernel,
        out_type=jax.ShapeDtypeStruct((N_TILES, N_ROWS, ROW_W), jnp.float32),
        mesh=mesh,
        compiler_params=pltpu.CompilerParams(needs_layout_passes=False),
        scratch_types=[
            pltpu.VMEM((CHUNK,), jnp.int32),
            pltpu.VMEM((N_ROWS, ROW_W), jnp.float32),
            pltpu.SemaphoreType.DMA,
        ],
    )
    def sc_hist(idxa_hbm, idxb_hbm, part_hbm, idx_v, hist_v, sem):
        wid = lax.axis_index("c") * 16 + lax.axis_index("s")

        cpa = pltpu.async_copy(idxa_hbm.at[pl.ds(wid * HALF, HALF)],
                               idx_v.at[pl.ds(0, HALF)], sem)
        cpb = pltpu.async_copy(idxb_hbm.at[pl.ds(wid * HALF, HALF)],
                               idx_v.at[pl.ds(HALF, HALF)], sem)

        zeros = jnp.zeros((16,), jnp.float32)

        def zero_body(i, _):
            hist_v[i >> 3, pl.ds((i & 7) * 16, 16)] = zeros
            return 0

        lax.fori_loop(0, N_ROWS * 8, zero_body, 0, unroll=8)

        cpa.wait()
        cpb.wait()

        ones = jnp.full((16,), 1.0, jnp.float32)

        def acc_body(i, _):
            v = idx_v[pl.ds(i * 16, 16)]
            plsc.addupdate_scatter(hist_v, [v >> 7, v & 127], ones)
            return 0

        lax.fori_loop(0, VECS, acc_body, 0, unroll=10)

        pltpu.sync_copy(hist_v, part_hbm.at[wid])

    return sc_hist


def _tc_combine_body(part_ref, out_ref):
    out_ref[...] = 1.0 / (jnp.sum(part_ref[...], axis=0) + 4.0)


_SC_HIST = _make_sc_hist()

_TC_COMBINE = pl.pallas_call(
    _tc_combine_body,
    out_shape=jax.ShapeDtypeStruct((N_ROWS, ROW_W), jnp.float32),
)


def kernel(X, ref_a, ref_b):
    del X
    parts = _SC_HIST(ref_a.reshape(-1), ref_b.reshape(-1))
    hist = _TC_COMBINE(parts)
    return hist.reshape(PAD_BINS)[:N_BINS]


# speedup vs baseline: 1.3660x; 1.1845x over previous
import functools

import jax
import jax.numpy as jnp
from jax import lax
from jax.experimental import pallas as pl
from jax.experimental.pallas import tpu as pltpu
from jax.experimental.pallas import tpu_sc as plsc

N_BINS = 10000
ROW_W = 128
N_ROWS = 80
PAD_BINS = N_ROWS * ROW_W
N_TILES = 32
E_TOTAL = 8 * 80000
CHUNK = E_TOTAL // N_TILES
HALF = CHUNK // 2
VECS = CHUNK // 16


def _make_sc_hist():
    mesh = plsc.VectorSubcoreMesh(core_axis_name="c", subcore_axis_name="s")

    @functools.partial(
        pl.kernel,
        out_type=jax.ShapeDtypeStruct((N_TILES, N_ROWS, ROW_W), jnp.float32),
        mesh=mesh,
        compiler_params=pltpu.CompilerParams(needs_layout_passes=False),
        scratch_types=[
            pltpu.VMEM((CHUNK,), jnp.int32),
            pltpu.VMEM((N_ROWS, ROW_W), jnp.float32),
            pltpu.SemaphoreType.DMA,
        ],
    )
    def sc_hist(idxa_hbm, idxb_hbm, part_hbm, idx_v, hist_v, sem):
        wid = lax.axis_index("c") * 16 + lax.axis_index("s")

        cpa = pltpu.async_copy(idxa_hbm.at[pl.ds(wid * HALF, HALF)],
                               idx_v.at[pl.ds(0, HALF)], sem)
        cpb = pltpu.async_copy(idxb_hbm.at[pl.ds(wid * HALF, HALF)],
                               idx_v.at[pl.ds(HALF, HALF)], sem)

        zeros = jnp.zeros((16,), jnp.float32)

        def zero_body(i, _):
            hist_v[i >> 3, pl.ds((i & 7) * 16, 16)] = zeros
            return 0

        lax.fori_loop(0, N_ROWS * 8, zero_body, 0, unroll=8)

        cpa.wait()
        cpb.wait()

        ones = jnp.full((16,), 1.0, jnp.float32)

        @plsc.parallel_loop(0, CHUNK, step=16, unroll=8)
        def acc_body(i):
            v = idx_v[pl.ds(i, 16)]
            plsc.addupdate_scatter(hist_v, [v >> 7, v & 127], ones)

        pltpu.sync_copy(hist_v, part_hbm.at[wid])

    return sc_hist


def _tc_combine_body(part_ref, out_ref):
    out_ref[...] = 1.0 / (jnp.sum(part_ref[...], axis=0) + 4.0)


_SC_HIST = _make_sc_hist()

_TC_COMBINE = pl.pallas_call(
    _tc_combine_body,
    out_shape=jax.ShapeDtypeStruct((N_ROWS, ROW_W), jnp.float32),
)


def kernel(X, ref_a, ref_b):
    del X
    parts = _SC_HIST(ref_a.reshape(-1), ref_b.reshape(-1))
    hist = _TC_COMBINE(parts)
    return hist.reshape(PAD_BINS)[:N_BINS]


# speedup vs baseline: 1.4258x; 1.0438x over previous
import functools

import jax
import jax.numpy as jnp
from jax import lax
from jax.experimental import pallas as pl
from jax.experimental.pallas import tpu as pltpu
from jax.experimental.pallas import tpu_sc as plsc

N_BINS = 10000
ROW_W = 128
N_ROWS = 80
PAD_BINS = N_ROWS * ROW_W
N_TILES = 32
N_REL = 4
E_PER_REL = 80000
LANE_BLKS = E_PER_REL // 128
BASE_BLKS = LANE_BLKS // N_TILES
EXTRA_TILES = LANE_BLKS - BASE_BLKS * N_TILES
BASE_COLS = BASE_BLKS * 128
MAX_COLS = BASE_COLS + 128


def _make_sc_hist():
    mesh = plsc.VectorSubcoreMesh(core_axis_name="c", subcore_axis_name="s")

    @functools.partial(
        pl.kernel,
        out_type=jax.ShapeDtypeStruct((N_TILES, N_ROWS, ROW_W), jnp.float32),
        mesh=mesh,
        compiler_params=pltpu.CompilerParams(needs_layout_passes=False),
        scratch_types=[
            pltpu.VMEM((N_REL, MAX_COLS), jnp.int32),
            pltpu.VMEM((N_REL, MAX_COLS), jnp.int32),
            pltpu.VMEM((N_ROWS, ROW_W), jnp.float32),
            pltpu.SemaphoreType.DMA,
        ],
    )
    def sc_hist(idxa_hbm, idxb_hbm, part_hbm, idxa_v, idxb_v, hist_v, sem):
        wid = lax.axis_index("c") * 16 + lax.axis_index("s")

        has_extra = wid < EXTRA_TILES
        base_off = pl.multiple_of(wid * BASE_COLS, 128)
        extra_off = pl.multiple_of((BASE_BLKS * N_TILES + wid) * 128, 128)

        cps = [
            pltpu.async_copy(idxa_hbm.at[:, pl.ds(base_off, BASE_COLS)],
                             idxa_v.at[:, pl.ds(0, BASE_COLS)], sem),
            pltpu.async_copy(idxb_hbm.at[:, pl.ds(base_off, BASE_COLS)],
                             idxb_v.at[:, pl.ds(0, BASE_COLS)], sem),
        ]

        @pl.when(has_extra)
        def _():
            pltpu.async_copy(idxa_hbm.at[:, pl.ds(extra_off, 128)],
                             idxa_v.at[:, pl.ds(BASE_COLS, 128)], sem).wait()
            pltpu.async_copy(idxb_hbm.at[:, pl.ds(extra_off, 128)],
                             idxb_v.at[:, pl.ds(BASE_COLS, 128)], sem).wait()

        zeros = jnp.zeros((16,), jnp.float32)

        def zero_body(i, _):
            hist_v[i >> 3, pl.ds((i & 7) * 16, 16)] = zeros
            return 0

        lax.fori_loop(0, N_ROWS * 8, zero_body, 0, unroll=8)

        for cp in cps:
            cp.wait()

        ones = jnp.full((16,), 1.0, jnp.float32)
        ncols = jnp.where(has_extra, MAX_COLS, BASE_COLS)

        for src_v in (idxa_v, idxb_v):
            for r in range(N_REL):

                @plsc.parallel_loop(0, ncols, step=16, unroll=8)
                def acc_body(i):
                    v = src_v[r, pl.ds(i, 16)]
                    plsc.addupdate_scatter(hist_v, [v >> 7, v & 127], ones)

        pltpu.sync_copy(hist_v, part_hbm.at[wid])

    return sc_hist


def _tc_combine_body(part_ref, out_ref):
    out_ref[...] = 1.0 / (jnp.sum(part_ref[...], axis=0) + 4.0)


_SC_HIST = _make_sc_hist()

_TC_COMBINE = pl.pallas_call(
    _tc_combine_body,
    out_shape=jax.ShapeDtypeStruct((N_ROWS, ROW_W), jnp.float32),
)


def kernel(X, ref_a, ref_b):
    del X
    parts = _SC_HIST(ref_a, ref_b)
    hist = _TC_COMBINE(parts)
    return hist.reshape(PAD_BINS)[:N_BINS]


# speedup vs baseline: 1.4454x; 1.0137x over previous
import functools

import jax
import jax.numpy as jnp
from jax import lax
from jax.experimental import pallas as pl
from jax.experimental.pallas import tpu as pltpu
from jax.experimental.pallas import tpu_sc as plsc

N_BINS = 10000
ROW_W = 128
N_ROWS = 80
PAD_BINS = N_ROWS * ROW_W
N_TILES = 32
N_REL = 4
E_PER_REL = 80000
LANE_BLKS = E_PER_REL // 128
BASE_BLKS = LANE_BLKS // N_TILES
EXTRA_TILES = LANE_BLKS - BASE_BLKS * N_TILES
BASE_COLS = BASE_BLKS * 128
MAX_COLS = BASE_COLS + 128


def _make_sc_hist():
    mesh = plsc.VectorSubcoreMesh(core_axis_name="c", subcore_axis_name="s")

    @functools.partial(
        pl.kernel,
        out_type=jax.ShapeDtypeStruct((N_TILES, N_ROWS, ROW_W), jnp.float32),
        mesh=mesh,
        compiler_params=pltpu.CompilerParams(needs_layout_passes=False),
        scratch_types=[
            pltpu.VMEM((2 * N_REL, MAX_COLS), jnp.int32),
            pltpu.VMEM((N_ROWS, ROW_W), jnp.float32),
            pltpu.SemaphoreType.DMA,
        ],
    )
    def sc_hist(idxa_hbm, idxb_hbm, part_hbm, idx_v, hist_v, sem):
        wid = lax.axis_index("c") * 16 + lax.axis_index("s")

        has_extra = wid < EXTRA_TILES
        base_off = pl.multiple_of(wid * BASE_COLS, 128)
        extra_off = pl.multiple_of((BASE_BLKS * N_TILES + wid) * 128, 128)

        cps = [
            pltpu.async_copy(idxa_hbm.at[:, pl.ds(base_off, BASE_COLS)],
                             idx_v.at[pl.ds(0, N_REL), pl.ds(0, BASE_COLS)],
                             sem),
            pltpu.async_copy(idxb_hbm.at[:, pl.ds(base_off, BASE_COLS)],
                             idx_v.at[pl.ds(N_REL, N_REL),
                                      pl.ds(0, BASE_COLS)], sem),
        ]

        @pl.when(has_extra)
        def _():
            pltpu.async_copy(idxa_hbm.at[:, pl.ds(extra_off, 128)],
                             idx_v.at[pl.ds(0, N_REL),
                                      pl.ds(BASE_COLS, 128)], sem).wait()
            pltpu.async_copy(idxb_hbm.at[:, pl.ds(extra_off, 128)],
                             idx_v.at[pl.ds(N_REL, N_REL),
                                      pl.ds(BASE_COLS, 128)], sem).wait()

        zeros = jnp.zeros((16,), jnp.float32)

        def zero_body(i, _):
            hist_v[i >> 3, pl.ds((i & 7) * 16, 16)] = zeros
            return 0

        lax.fori_loop(0, N_ROWS * 8, zero_body, 0, unroll=4)

        for cp in cps:
            cp.wait()

        ones = jnp.full((16,), 1.0, jnp.float32)
        ncols = jnp.where(has_extra, MAX_COLS, BASE_COLS)

        @plsc.parallel_loop(0, ncols, step=16, unroll=2)
        def acc_body(i):
            for r in range(2 * N_REL):
                v = idx_v[r, pl.ds(i, 16)]
                plsc.addupdate_scatter(hist_v, [v >> 7, v & 127], ones)

        pltpu.sync_copy(hist_v, part_hbm.at[wid])

    return sc_hist


def _tc_combine_body(part_ref, out_ref):
    out_ref[...] = 1.0 / (jnp.sum(part_ref[...], axis=0) + 4.0)


_SC_HIST = _make_sc_hist()

_TC_COMBINE = pl.pallas_call(
    _tc_combine_body,
    out_shape=jax.ShapeDtypeStruct((N_ROWS, ROW_W), jnp.float32),
)


def kernel(X, ref_a, ref_b):
    del X
    parts = _SC_HIST(ref_a, ref_b)
    hist = _TC_COMBINE(parts)
    return hist.reshape(PAD_BINS)[:N_BINS]
